# Initial kernel scaffold; baseline (speedup 1.0000x reference)
import functools

import jax
import jax.numpy as jnp
from jax import lax
from jax.experimental import pallas as pl
from jax.experimental.pallas import tpu as pltpu
from jax.experimental.pallas import tpu_sc as plsc

D_CONTENT = 128
NUM_TOOLS = 100000
NUM_LLM = 1000
TOK_DIM = 64
ID_DIM = 64
B = 16384
L = 20

NW = 32
BPW = B // NW
C = 32
NCH = BPW // C
G = C * L
NG = G // 128


def _sc_body(tidx_hbm, mask_hbm, lidx_hbm, tool_hbm, llm_hbm,
             tm_out, llm_out,
             idx_v, rows_v, mask_v, out_v, lidx_v, lrows_v, sem, lsem):
    cid = lax.axis_index("c")
    sid = lax.axis_index("s")
    wid = sid * 2 + cid
    base = wid * BPW

    def chunk_body(t, carry):
        eb = base + t * C
        ib = eb * L
        pltpu.sync_copy(tidx_hbm.at[pl.ds(ib // 128, NG)], idx_v)
        pltpu.sync_copy(mask_hbm.at[pl.ds(ib, G)], mask_v)
        pltpu.sync_copy(lidx_hbm.at[pl.ds(eb, C)], lidx_v)
        lcp = pltpu.async_copy(llm_hbm.at[lidx_v], lrows_v, lsem)
        cps = [
            pltpu.async_copy(tool_hbm.at[idx_v.at[j]],
                             rows_v.at[pl.ds(j * 128, 128)], sem)
            for j in range(NG)
        ]
        for cp in cps:
            cp.wait()
        lcp.wait()
        pltpu.sync_copy(lrows_v, llm_out.at[pl.ds(eb, C)])

        def elem_body(c, carry2):
            rb = c * L
            msum = jnp.zeros((16,), jnp.float32)
            acc = [jnp.zeros((16,), jnp.float32) for _ in range(4)]
            for l in range(L):
                r = rb + l
                m = plsc.load_gather(mask_v, [jnp.full((16,), r, jnp.int32)])
                msum = msum + m
                for k in range(4):
                    acc[k] = acc[k] + m * rows_v[r, pl.ds(k * 16, 16)]
            inv = 1.0 / (msum + 1e-8)
            for k in range(4):
                out_v[c, pl.ds(k * 16, 16)] = acc[k] * inv
            return carry2

        lax.fori_loop(0, C, elem_body, 0)
        pltpu.sync_copy(out_v, tm_out.at[pl.ds(eb, C)])
        return carry

    lax.fori_loop(0, NCH, chunk_body, 0)


@jax.jit
def _sc_call(tidx2d, mask_flat, lidx, emb_tool, emb_llm):
    mesh = plsc.VectorSubcoreMesh(core_axis_name="c", subcore_axis_name="s")
    fn = functools.partial(
        pl.kernel,
        mesh=mesh,
        out_type=[
            jax.ShapeDtypeStruct((B, ID_DIM), jnp.float32),
            jax.ShapeDtypeStruct((B, ID_DIM), jnp.float32),
        ],
        scratch_types=[
            pltpu.VMEM((NG, 128), jnp.int32),
            pltpu.VMEM((G, ID_DIM), jnp.float32),
            pltpu.VMEM((G,), jnp.float32),
            pltpu.VMEM((C, ID_DIM), jnp.float32),
            pltpu.VMEM((C,), jnp.int32),
            pltpu.VMEM((C, ID_DIM), jnp.float32),
            pltpu.SemaphoreType.DMA,
            pltpu.SemaphoreType.DMA,
        ],
    )(_sc_body)
    return fn(tidx2d, mask_flat, lidx, emb_tool, emb_llm)


def _tc_body(a_ref, llm_ref, tm_ref, wc_ref, wl_ref, wt_ref, o_ref):
    e = jnp.dot(a_ref[...], wc_ref[...], preferred_element_type=jnp.float32)
    e = e + jnp.dot(llm_ref[...], wl_ref[...], preferred_element_type=jnp.float32)
    e = e + jnp.dot(tm_ref[...], wt_ref[...], preferred_element_type=jnp.float32)
    n = jnp.sqrt(jnp.sum(e * e, axis=1, keepdims=True))
    o_ref[...] = e / jnp.maximum(n, 1e-12)


@jax.jit
def _tc_call(a, llm_e, tm, wc_t, wl_t, wt_t):
    R = 1024
    grid = (B // R,)
    return pl.pallas_call(
        _tc_body,
        grid=grid,
        in_specs=[
            pl.BlockSpec((R, D_CONTENT), lambda i: (i, 0)),
            pl.BlockSpec((R, ID_DIM), lambda i: (i, 0)),
            pl.BlockSpec((R, ID_DIM), lambda i: (i, 0)),
            pl.BlockSpec((D_CONTENT, TOK_DIM), lambda i: (0, 0)),
            pl.BlockSpec((ID_DIM, TOK_DIM), lambda i: (0, 0)),
            pl.BlockSpec((ID_DIM, TOK_DIM), lambda i: (0, 0)),
        ],
        out_specs=pl.BlockSpec((R, TOK_DIM), lambda i: (i, 0)),
        out_shape=jax.ShapeDtypeStruct((B, TOK_DIM), jnp.float32),
    )(a, llm_e, tm, wc_t, wl_t, wt_t)


def kernel(A_content, tool_idx_pad, tool_mask, llm_idx, emb_tool, emb_llm,
           W_content, W_ids):
    tidx2d = tool_idx_pad.astype(jnp.int32).reshape(B * L // 128, 128)
    mask_flat = tool_mask.reshape(B * L)
    lidx = llm_idx.astype(jnp.int32)
    tm, llm_e = _sc_call(tidx2d, mask_flat, lidx, emb_tool, emb_llm)
    return _tc_call(A_content, llm_e, tm, W_content.T,
                    W_ids[:, :ID_DIM].T, W_ids[:, ID_DIM:].T)


# speedup vs baseline: 5.8615x; 5.8615x over previous
import functools

import jax
import jax.numpy as jnp
from jax import lax
from jax.experimental import pallas as pl
from jax.experimental.pallas import tpu as pltpu
from jax.experimental.pallas import tpu_sc as plsc

D_CONTENT = 128
NUM_TOOLS = 100000
NUM_LLM = 1000
TOK_DIM = 64
ID_DIM = 64
B = 16384
L = 20

NW = 32
BPW = B // NW
C = 32
NCH = BPW // C
G = C * L
NG = G // 128


def _sc_body(tidx_hbm, mask_hbm, lidx_hbm, tool_hbm, llm_hbm,
             tm_out, llm_out,
             idx_v, rows_v, mask_v, out_v, lidx_v, lrows_v, sem, lsem):
    cid = lax.axis_index("c")
    sid = lax.axis_index("s")
    wid = sid * 2 + cid
    base = wid * BPW

    def chunk_body(t, carry):
        eb = base + t * C
        ib = eb * L
        for j in range(NG):
            pltpu.sync_copy(tidx_hbm.at[pl.ds(ib + j * 128, 128)], idx_v.at[j])
        pltpu.sync_copy(mask_hbm.at[pl.ds(ib, G)], mask_v.at[pl.ds(0, G)])
        pltpu.sync_copy(lidx_hbm.at[pl.ds(eb, C)], lidx_v)
        lcp = pltpu.async_copy(llm_hbm.at[lidx_v], lrows_v, lsem)
        cps = [
            pltpu.async_copy(tool_hbm.at[idx_v.at[j]],
                             rows_v.at[pl.ds(j * 128, 128)], sem)
            for j in range(NG)
        ]
        for cp in cps:
            cp.wait()
        lcp.wait()
        pltpu.sync_copy(lrows_v, llm_out.at[pl.ds(eb, C)])

        def elem_body(c, carry2):
            rb = c * L
            mv0 = mask_v[pl.ds(rb, 16)]
            mv1 = mask_v[pl.ds(rb + 16, 16)]
            msum = jnp.zeros((16,), jnp.float32)
            acc = [jnp.zeros((16,), jnp.float32) for _ in range(4)]
            for l in range(L):
                r = rb + l
                m = jnp.full((16,), mv0[l] if l < 16 else mv1[l - 16])
                msum = msum + m
                for k in range(4):
                    acc[k] = acc[k] + m * rows_v[r, pl.ds(k * 16, 16)]
            inv = 1.0 / (msum + 1e-8)
            for k in range(4):
                out_v[c, pl.ds(k * 16, 16)] = acc[k] * inv
            return carry2

        lax.fori_loop(0, C, elem_body, 0)
        pltpu.sync_copy(out_v, tm_out.at[pl.ds(eb, C)])
        return carry

    lax.fori_loop(0, NCH, chunk_body, 0)


@jax.jit
def _sc_call(tidx_flat, mask_flat, lidx, emb_tool, emb_llm):
    mesh = plsc.VectorSubcoreMesh(core_axis_name="c", subcore_axis_name="s")
    fn = functools.partial(
        pl.kernel,
        mesh=mesh,
        compiler_params=pltpu.CompilerParams(use_tc_tiling_on_sc=False),
        out_type=[
            jax.ShapeDtypeStruct((B, ID_DIM), jnp.float32),
            jax.ShapeDtypeStruct((B, ID_DIM), jnp.float32),
        ],
        scratch_types=[
            pltpu.VMEM((NG, 128), jnp.int32),
            pltpu.VMEM((G, ID_DIM), jnp.float32),
            pltpu.VMEM((G + 16,), jnp.float32),
            pltpu.VMEM((C, ID_DIM), jnp.float32),
            pltpu.VMEM((C,), jnp.int32),
            pltpu.VMEM((C, ID_DIM), jnp.float32),
            pltpu.SemaphoreType.DMA,
            pltpu.SemaphoreType.DMA,
        ],
    )(_sc_body)
    return fn(tidx_flat, mask_flat, lidx, emb_tool, emb_llm)


def _tc_body(a_ref, llm_ref, tm_ref, wc_ref, wl_ref, wt_ref, o_ref):
    e = jnp.dot(a_ref[...], wc_ref[...], preferred_element_type=jnp.float32)
    e = e + jnp.dot(llm_ref[...], wl_ref[...], preferred_element_type=jnp.float32)
    e = e + jnp.dot(tm_ref[...], wt_ref[...], preferred_element_type=jnp.float32)
    n = jnp.sqrt(jnp.sum(e * e, axis=1, keepdims=True))
    o_ref[...] = e / jnp.maximum(n, 1e-12)


@jax.jit
def _tc_call(a, llm_e, tm, wc_t, wl_t, wt_t):
    R = 1024
    grid = (B // R,)
    return pl.pallas_call(
        _tc_body,
        grid=grid,
        in_specs=[
            pl.BlockSpec((R, D_CONTENT), lambda i: (i, 0)),
            pl.BlockSpec((R, ID_DIM), lambda i: (i, 0)),
            pl.BlockSpec((R, ID_DIM), lambda i: (i, 0)),
            pl.BlockSpec((D_CONTENT, TOK_DIM), lambda i: (0, 0)),
            pl.BlockSpec((ID_DIM, TOK_DIM), lambda i: (0, 0)),
            pl.BlockSpec((ID_DIM, TOK_DIM), lambda i: (0, 0)),
        ],
        out_specs=pl.BlockSpec((R, TOK_DIM), lambda i: (i, 0)),
        out_shape=jax.ShapeDtypeStruct((B, TOK_DIM), jnp.float32),
    )(a, llm_e, tm, wc_t, wl_t, wt_t)


def kernel(A_content, tool_idx_pad, tool_mask, llm_idx, emb_tool, emb_llm,
           W_content, W_ids):
    tidx_flat = tool_idx_pad.astype(jnp.int32).reshape(B * L)
    mask_flat = tool_mask.reshape(B * L)
    lidx = llm_idx.astype(jnp.int32)
    tm, llm_e = _sc_call(tidx_flat, mask_flat, lidx, emb_tool, emb_llm)
    return _tc_call(A_content, llm_e, tm, W_content.T,
                    W_ids[:, :ID_DIM].T, W_ids[:, ID_DIM:].T)


# speedup vs baseline: 8.5801x; 1.4638x over previous
import functools

import jax
import jax.numpy as jnp
from jax import lax
from jax.experimental import pallas as pl
from jax.experimental.pallas import tpu as pltpu
from jax.experimental.pallas import tpu_sc as plsc

D_CONTENT = 128
NUM_TOOLS = 100000
NUM_LLM = 1000
TOK_DIM = 64
ID_DIM = 64
B = 16384
L = 20

NW = 32
BPW = B // NW
C = 32
NCH = BPW // C
G = C * L
NG = G // 128


def _sc_body(tidx_hbm, mask_hbm, lidx_hbm, tool_hbm, llm_hbm,
             tm_out, llm_out,
             idxb, maskb, lidxb, rows0, rows1, lrows0, lrows1,
             out0, out1, sem0, sem1, lsem0, lsem1, ssem):
    cid = lax.axis_index("c")
    sid = lax.axis_index("s")
    wid = sid * 2 + cid
    base = wid * BPW
    ib0 = base * L

    cp1 = pltpu.async_copy(tidx_hbm.at[pl.ds(ib0, BPW * L)], idxb, ssem)
    cp2 = pltpu.async_copy(mask_hbm.at[pl.ds(ib0, BPW * L)],
                           maskb.at[pl.ds(0, BPW * L)], ssem)
    cp3 = pltpu.async_copy(lidx_hbm.at[pl.ds(base, BPW)], lidxb, ssem)
    cp1.wait()
    cp2.wait()
    cp3.wait()

    rows = [rows0, rows1]
    lrows = [lrows0, lrows1]
    outs = [out0, out1]
    sems = [sem0, sem1]
    lsems = [lsem0, lsem1]

    def fire(t, b):
        for j in range(NG):
            pltpu.async_copy(
                tool_hbm.at[idxb.at[pl.ds(t * G + j * 128, 128)]],
                rows[b].at[pl.ds(j * 128, 128)], sems[b])
        pltpu.async_copy(llm_hbm.at[lidxb.at[pl.ds(t * C, C)]],
                         lrows[b], lsems[b])

    def drain(b):
        pltpu.make_async_copy(tool_hbm.at[pl.ds(0, G)], rows[b],
                              sems[b]).wait()
        pltpu.make_async_copy(llm_hbm.at[pl.ds(0, C)], lrows[b],
                              lsems[b]).wait()

    def compute(t, b):
        mb0 = t * G

        def elem_body(c, carry):
            rb = c * L
            mv0 = maskb[pl.ds(mb0 + rb, 16)]
            mv1 = maskb[pl.ds(mb0 + rb + 16, 16)]
            msum = jnp.zeros((16,), jnp.float32)
            acc = [jnp.zeros((16,), jnp.float32) for _ in range(4)]
            for l in range(L):
                r = rb + l
                m = jnp.full((16,), mv0[l] if l < 16 else mv1[l - 16])
                msum = msum + m
                for k in range(4):
                    acc[k] = acc[k] + m * rows[b][r, pl.ds(k * 16, 16)]
            inv = 1.0 / (msum + 1e-8)
            for k in range(4):
                outs[b][c, pl.ds(k * 16, 16)] = acc[k] * inv
            return carry

        lax.fori_loop(0, C, elem_body, 0)
        eb = base + t * C
        pltpu.sync_copy(outs[b], tm_out.at[pl.ds(eb, C)])
        pltpu.sync_copy(lrows[b], llm_out.at[pl.ds(eb, C)])

    fire(0, 0)

    def pair_body(g2, carry):
        t0 = 2 * g2
        fire(t0 + 1, 1)
        drain(0)
        compute(t0, 0)

        @pl.when(g2 < NCH // 2 - 1)
        def _():
            fire(t0 + 2, 0)

        drain(1)
        compute(t0 + 1, 1)
        return carry

    lax.fori_loop(0, NCH // 2, pair_body, 0)


@jax.jit
def _sc_call(tidx_flat, mask_flat, lidx, emb_tool, emb_llm):
    mesh = plsc.VectorSubcoreMesh(core_axis_name="c", subcore_axis_name="s")
    fn = functools.partial(
        pl.kernel,
        mesh=mesh,
        compiler_params=pltpu.CompilerParams(use_tc_tiling_on_sc=False),
        out_type=[
            jax.ShapeDtypeStruct((B, ID_DIM), jnp.float32),
            jax.ShapeDtypeStruct((B, ID_DIM), jnp.float32),
        ],
        scratch_types=[
            pltpu.VMEM((BPW * L,), jnp.int32),
            pltpu.VMEM((BPW * L + 16,), jnp.float32),
            pltpu.VMEM((BPW,), jnp.int32),
            pltpu.VMEM((G, ID_DIM), jnp.float32),
            pltpu.VMEM((G, ID_DIM), jnp.float32),
            pltpu.VMEM((C, ID_DIM), jnp.float32),
            pltpu.VMEM((C, ID_DIM), jnp.float32),
            pltpu.VMEM((C, ID_DIM), jnp.float32),
            pltpu.VMEM((C, ID_DIM), jnp.float32),
            pltpu.SemaphoreType.DMA,
            pltpu.SemaphoreType.DMA,
            pltpu.SemaphoreType.DMA,
            pltpu.SemaphoreType.DMA,
            pltpu.SemaphoreType.DMA,
        ],
    )(_sc_body)
    return fn(tidx_flat, mask_flat, lidx, emb_tool, emb_llm)


def _tc_body(a_ref, llm_ref, tm_ref, wc_ref, wl_ref, wt_ref, o_ref):
    e = jnp.dot(a_ref[...], wc_ref[...], preferred_element_type=jnp.float32)
    e = e + jnp.dot(llm_ref[...], wl_ref[...], preferred_element_type=jnp.float32)
    e = e + jnp.dot(tm_ref[...], wt_ref[...], preferred_element_type=jnp.float32)
    n = jnp.sqrt(jnp.sum(e * e, axis=1, keepdims=True))
    o_ref[...] = e / jnp.maximum(n, 1e-12)


@jax.jit
def _tc_call(a, llm_e, tm, wc_t, wl_t, wt_t):
    R = 1024
    grid = (B // R,)
    return pl.pallas_call(
        _tc_body,
        grid=grid,
        in_specs=[
            pl.BlockSpec((R, D_CONTENT), lambda i: (i, 0)),
            pl.BlockSpec((R, ID_DIM), lambda i: (i, 0)),
            pl.BlockSpec((R, ID_DIM), lambda i: (i, 0)),
            pl.BlockSpec((D_CONTENT, TOK_DIM), lambda i: (0, 0)),
            pl.BlockSpec((ID_DIM, TOK_DIM), lambda i: (0, 0)),
            pl.BlockSpec((ID_DIM, TOK_DIM), lambda i: (0, 0)),
        ],
        out_specs=pl.BlockSpec((R, TOK_DIM), lambda i: (i, 0)),
        out_shape=jax.ShapeDtypeStruct((B, TOK_DIM), jnp.float32),
    )(a, llm_e, tm, wc_t, wl_t, wt_t)


def kernel(A_content, tool_idx_pad, tool_mask, llm_idx, emb_tool, emb_llm,
           W_content, W_ids):
    tidx_flat = tool_idx_pad.astype(jnp.int32).reshape(B * L)
    mask_flat = tool_mask.reshape(B * L)
    lidx = llm_idx.astype(jnp.int32)
    tm, llm_e = _sc_call(tidx_flat, mask_flat, lidx, emb_tool, emb_llm)
    return _tc_call(A_content, llm_e, tm, W_content.T,
                    W_ids[:, :ID_DIM].T, W_ids[:, ID_DIM:].T)


# speedup vs baseline: 9.2448x; 1.0775x over previous
import functools

import jax
import jax.numpy as jnp
from jax import lax
from jax.experimental import pallas as pl
from jax.experimental.pallas import tpu as pltpu
from jax.experimental.pallas import tpu_sc as plsc

D_CONTENT = 128
NUM_TOOLS = 100000
NUM_LLM = 1000
TOK_DIM = 64
ID_DIM = 64
B = 16384
L = 20

NW = 32
BPW = B // NW
C = 32
NCH = BPW // C
G = C * L
NG = G // 128


def _sc_body(tidx_hbm, mask_hbm, lidx_hbm, tool_hbm, llm_hbm,
             ids_out,
             idxb, maskb, lidxb, rows0, rows1, lrows0, lrows1,
             out0, out1, sem0, sem1, lsem0, lsem1, ssem):
    cid = lax.axis_index("c")
    sid = lax.axis_index("s")
    wid = sid * 2 + cid
    base = wid * BPW
    ib0 = base * L

    cp1 = pltpu.async_copy(tidx_hbm.at[pl.ds(ib0, BPW * L)], idxb, ssem)
    cp2 = pltpu.async_copy(mask_hbm.at[pl.ds(ib0, BPW * L)],
                           maskb.at[pl.ds(0, BPW * L)], ssem)
    cp3 = pltpu.async_copy(lidx_hbm.at[pl.ds(base, BPW)], lidxb, ssem)
    cp1.wait()
    cp2.wait()
    cp3.wait()

    rows = [rows0, rows1]
    lrows = [lrows0, lrows1]
    outs = [out0, out1]
    sems = [sem0, sem1]
    lsems = [lsem0, lsem1]

    def fire(t, b):
        for j in range(NG):
            pltpu.async_copy(
                tool_hbm.at[idxb.at[pl.ds(t * G + j * 128, 128)]],
                rows[b].at[pl.ds(j * 128, 128)], sems[b])
        pltpu.async_copy(llm_hbm.at[lidxb.at[pl.ds(t * C, C)]],
                         lrows[b], lsems[b])

    def drain(b):
        pltpu.make_async_copy(tool_hbm.at[pl.ds(0, G)], rows[b],
                              sems[b]).wait()
        pltpu.make_async_copy(llm_hbm.at[pl.ds(0, C)], lrows[b],
                              lsems[b]).wait()

    def compute(t, b):
        mb0 = t * G

        def elem_body(c, carry):
            rb = c * L
            mv0 = maskb[pl.ds(mb0 + rb, 16)]
            mv1 = maskb[pl.ds(mb0 + rb + 16, 16)]
            msum = jnp.zeros((16,), jnp.float32)
            acc = [jnp.zeros((16,), jnp.float32) for _ in range(4)]
            for l in range(L):
                r = rb + l
                m = jnp.full((16,), mv0[l] if l < 16 else mv1[l - 16])
                msum = msum + m
                for k in range(4):
                    acc[k] = acc[k] + m * rows[b][r, pl.ds(k * 16, 16)]
            inv = 1.0 / (msum + 1e-8)
            for k in range(4):
                outs[b][c, pl.ds(ID_DIM + k * 16, 16)] = acc[k] * inv
            for k in range(4):
                outs[b][c, pl.ds(k * 16, 16)] = lrows[b][c, pl.ds(k * 16, 16)]
            return carry

        lax.fori_loop(0, C, elem_body, 0)
        eb = base + t * C
        pltpu.sync_copy(outs[b], ids_out.at[pl.ds(eb, C)])

    fire(0, 0)

    def pair_body(g2, carry):
        t0 = 2 * g2
        fire(t0 + 1, 1)
        drain(0)
        compute(t0, 0)

        @pl.when(g2 < NCH // 2 - 1)
        def _():
            fire(t0 + 2, 0)

        drain(1)
        compute(t0 + 1, 1)
        return carry

    lax.fori_loop(0, NCH // 2, pair_body, 0)


@jax.jit
def _sc_call(tidx_flat, mask_flat, lidx, emb_tool, emb_llm):
    mesh = plsc.VectorSubcoreMesh(core_axis_name="c", subcore_axis_name="s")
    fn = functools.partial(
        pl.kernel,
        mesh=mesh,
        compiler_params=pltpu.CompilerParams(use_tc_tiling_on_sc=False),
        out_type=jax.ShapeDtypeStruct((B, 2 * ID_DIM), jnp.float32),
        scratch_types=[
            pltpu.VMEM((BPW * L,), jnp.int32),
            pltpu.VMEM((BPW * L + 16,), jnp.float32),
            pltpu.VMEM((BPW,), jnp.int32),
            pltpu.VMEM((G, ID_DIM), jnp.float32),
            pltpu.VMEM((G, ID_DIM), jnp.float32),
            pltpu.VMEM((C, ID_DIM), jnp.float32),
            pltpu.VMEM((C, ID_DIM), jnp.float32),
            pltpu.VMEM((C, 2 * ID_DIM), jnp.float32),
            pltpu.VMEM((C, 2 * ID_DIM), jnp.float32),
            pltpu.SemaphoreType.DMA,
            pltpu.SemaphoreType.DMA,
            pltpu.SemaphoreType.DMA,
            pltpu.SemaphoreType.DMA,
            pltpu.SemaphoreType.DMA,
        ],
    )(_sc_body)
    return fn(tidx_flat, mask_flat, lidx, emb_tool, emb_llm)


def _mm_body(a_ref, wc_ref, o_ref):
    o_ref[...] = jnp.dot(a_ref[...], wc_ref[...],
                         preferred_element_type=jnp.float32)


@jax.jit
def _mm_call(a, wc_t):
    R = 2048
    return pl.pallas_call(
        _mm_body,
        grid=(B // R,),
        in_specs=[
            pl.BlockSpec((R, D_CONTENT), lambda i: (i, 0)),
            pl.BlockSpec((D_CONTENT, TOK_DIM), lambda i: (0, 0)),
        ],
        out_specs=pl.BlockSpec((R, TOK_DIM), lambda i: (i, 0)),
        out_shape=jax.ShapeDtypeStruct((B, TOK_DIM), jnp.float32),
    )(a, wc_t)


def _fin_body(e1_ref, ids_ref, wi_ref, o_ref):
    e = e1_ref[...] + jnp.dot(ids_ref[...], wi_ref[...],
                              preferred_element_type=jnp.float32)
    n = jnp.sqrt(jnp.sum(e * e, axis=1, keepdims=True))
    o_ref[...] = e / jnp.maximum(n, 1e-12)


@jax.jit
def _fin_call(e1, ids, wi_t):
    R = 2048
    return pl.pallas_call(
        _fin_body,
        grid=(B // R,),
        in_specs=[
            pl.BlockSpec((R, TOK_DIM), lambda i: (i, 0)),
            pl.BlockSpec((R, 2 * ID_DIM), lambda i: (i, 0)),
            pl.BlockSpec((2 * ID_DIM, TOK_DIM), lambda i: (0, 0)),
        ],
        out_specs=pl.BlockSpec((R, TOK_DIM), lambda i: (i, 0)),
        out_shape=jax.ShapeDtypeStruct((B, TOK_DIM), jnp.float32),
    )(e1, ids, wi_t)


def kernel(A_content, tool_idx_pad, tool_mask, llm_idx, emb_tool, emb_llm,
           W_content, W_ids):
    tidx_flat = tool_idx_pad.astype(jnp.int32).reshape(B * L)
    mask_flat = tool_mask.reshape(B * L)
    lidx = llm_idx.astype(jnp.int32)
    ids = _sc_call(tidx_flat, mask_flat, lidx, emb_tool, emb_llm)
    e1 = _mm_call(A_content, W_content.T)
    return _fin_call(e1, ids, W_ids.T)


# speedup vs baseline: 9.3950x; 1.0163x over previous
import functools

import jax
import jax.numpy as jnp
from jax import lax
from jax.experimental import pallas as pl
from jax.experimental.pallas import tpu as pltpu
from jax.experimental.pallas import tpu_sc as plsc

D_CONTENT = 128
NUM_TOOLS = 100000
NUM_LLM = 1000
TOK_DIM = 64
ID_DIM = 64
B = 16384
L = 20

NW = 32
BPW = B // NW
C = 32
NCH = BPW // C
G = C * L
NG = G // 128


def _sc_body(tidx_hbm, mask_hbm, lidx_hbm, tool_hbm, llm_hbm,
             ids_out,
             idxb, maskb, lidxb, rows0, rows1, lrows0, lrows1,
             out0, out1, sem0, sem1, lsem0, lsem1, ssem):
    cid = lax.axis_index("c")
    sid = lax.axis_index("s")
    wid = sid * 2 + cid
    base = wid * BPW
    ib0 = base * L

    cp1 = pltpu.async_copy(tidx_hbm.at[pl.ds(ib0, BPW * L)], idxb, ssem)
    cp2 = pltpu.async_copy(mask_hbm.at[pl.ds(ib0, BPW * L)],
                           maskb.at[pl.ds(0, BPW * L)], ssem)
    cp3 = pltpu.async_copy(lidx_hbm.at[pl.ds(base, BPW)], lidxb, ssem)
    cp1.wait()
    cp2.wait()
    cp3.wait()

    rows = [rows0, rows1]
    lrows = [lrows0, lrows1]
    outs = [out0, out1]
    sems = [sem0, sem1]
    lsems = [lsem0, lsem1]

    def fire(t, b):
        for j in range(NG):
            pltpu.async_copy(
                tool_hbm.at[idxb.at[pl.ds(t * G + j * 128, 128)]],
                rows[b].at[pl.ds(j * 128, 128)], sems[b])
        pltpu.async_copy(llm_hbm.at[lidxb.at[pl.ds(t * C, C)]],
                         lrows[b], lsems[b])

    def drain(b):
        pltpu.make_async_copy(tool_hbm.at[pl.ds(0, G)], rows[b],
                              sems[b]).wait()
        pltpu.make_async_copy(llm_hbm.at[pl.ds(0, C)], lrows[b],
                              lsems[b]).wait()

    def compute(t, b):
        mb0 = t * G

        def elem_body(c, carry):
            rb = c * L
            mv0 = maskb[pl.ds(mb0 + rb, 16)]
            mv1 = maskb[pl.ds(mb0 + rb + 16, 16)]
            msum = jnp.zeros((16,), jnp.float32)
            acc = [jnp.zeros((16,), jnp.float32) for _ in range(4)]
            for l in range(L):
                r = rb + l
                m = jnp.full((16,), mv0[l] if l < 16 else mv1[l - 16])
                msum = msum + m
                for k in range(4):
                    acc[k] = acc[k] + m * rows[b][r, pl.ds(k * 16, 16)]
            inv = 1.0 / (msum + 1e-8)
            for k in range(4):
                outs[b][c, pl.ds(k * 16, 16)] = acc[k] * inv
            return carry

        lax.fori_loop(0, C, elem_body, 0)
        eb = base + t * C
        pltpu.sync_copy(lrows[b],
                        ids_out.at[pl.ds(eb, C), pl.ds(0, ID_DIM)])
        pltpu.sync_copy(outs[b],
                        ids_out.at[pl.ds(eb, C), pl.ds(ID_DIM, ID_DIM)])

    fire(0, 0)

    def pair_body(g2, carry):
        t0 = 2 * g2
        fire(t0 + 1, 1)
        drain(0)
        compute(t0, 0)

        @pl.when(g2 < NCH // 2 - 1)
        def _():
            fire(t0 + 2, 0)

        drain(1)
        compute(t0 + 1, 1)
        return carry

    lax.fori_loop(0, NCH // 2, pair_body, 0)


@jax.jit
def _sc_call(tidx_flat, mask_flat, lidx, emb_tool, emb_llm):
    mesh = plsc.VectorSubcoreMesh(core_axis_name="c", subcore_axis_name="s")
    fn = functools.partial(
        pl.kernel,
        mesh=mesh,
        compiler_params=pltpu.CompilerParams(use_tc_tiling_on_sc=False),
        out_type=jax.ShapeDtypeStruct((B, 2 * ID_DIM), jnp.float32),
        scratch_types=[
            pltpu.VMEM((BPW * L,), jnp.int32),
            pltpu.VMEM((BPW * L + 16,), jnp.float32),
            pltpu.VMEM((BPW,), jnp.int32),
            pltpu.VMEM((G, ID_DIM), jnp.float32),
            pltpu.VMEM((G, ID_DIM), jnp.float32),
            pltpu.VMEM((C, ID_DIM), jnp.float32),
            pltpu.VMEM((C, ID_DIM), jnp.float32),
            pltpu.VMEM((C, ID_DIM), jnp.float32),
            pltpu.VMEM((C, ID_DIM), jnp.float32),
            pltpu.SemaphoreType.DMA,
            pltpu.SemaphoreType.DMA,
            pltpu.SemaphoreType.DMA,
            pltpu.SemaphoreType.DMA,
            pltpu.SemaphoreType.DMA,
        ],
    )(_sc_body)
    return fn(tidx_flat, mask_flat, lidx, emb_tool, emb_llm)


def _mm_body(a_ref, wc_ref, o_ref):
    o_ref[...] = jnp.dot(a_ref[...], wc_ref[...],
                         preferred_element_type=jnp.float32)


@jax.jit
def _mm_call(a, wc_t):
    R = 2048
    return pl.pallas_call(
        _mm_body,
        grid=(B // R,),
        in_specs=[
            pl.BlockSpec((R, D_CONTENT), lambda i: (i, 0)),
            pl.BlockSpec((D_CONTENT, TOK_DIM), lambda i: (0, 0)),
        ],
        out_specs=pl.BlockSpec((R, TOK_DIM), lambda i: (i, 0)),
        out_shape=jax.ShapeDtypeStruct((B, TOK_DIM), jnp.float32),
    )(a, wc_t)


def _fin_body(e1_ref, ids_ref, wi_ref, o_ref):
    e = e1_ref[...] + jnp.dot(ids_ref[...], wi_ref[...],
                              preferred_element_type=jnp.float32)
    n = jnp.sqrt(jnp.sum(e * e, axis=1, keepdims=True))
    o_ref[...] = e / jnp.maximum(n, 1e-12)


@jax.jit
def _fin_call(e1, ids, wi_t):
    R = 2048
    return pl.pallas_call(
        _fin_body,
        grid=(B // R,),
        in_specs=[
            pl.BlockSpec((R, TOK_DIM), lambda i: (i, 0)),
            pl.BlockSpec((R, 2 * ID_DIM), lambda i: (i, 0)),
            pl.BlockSpec((2 * ID_DIM, TOK_DIM), lambda i: (0, 0)),
        ],
        out_specs=pl.BlockSpec((R, TOK_DIM), lambda i: (i, 0)),
        out_shape=jax.ShapeDtypeStruct((B, TOK_DIM), jnp.float32),
    )(e1, ids, wi_t)


def kernel(A_content, tool_idx_pad, tool_mask, llm_idx, emb_tool, emb_llm,
           W_content, W_ids):
    tidx_flat = tool_idx_pad.astype(jnp.int32).reshape(B * L)
    mask_flat = tool_mask.reshape(B * L)
    lidx = llm_idx.astype(jnp.int32)
    ids = _sc_call(tidx_flat, mask_flat, lidx, emb_tool, emb_llm)
    e1 = _mm_call(A_content, W_content.T)
    return _fin_call(e1, ids, W_ids.T)


# speedup vs baseline: 10.3174x; 1.0982x over previous
import functools

import jax
import jax.numpy as jnp
from jax import lax
from jax.experimental import pallas as pl
from jax.experimental.pallas import tpu as pltpu
from jax.experimental.pallas import tpu_sc as plsc

D_CONTENT = 128
NUM_TOOLS = 100000
NUM_LLM = 1000
TOK_DIM = 64
ID_DIM = 64
B = 16384
L = 20

NW = 32
BPW = B // NW
C = 32
NCH = BPW // C
G = C * L
NG = G // 128


def _sc_body(tidx_hbm, mask_hbm, lidx_hbm, tool_hbm, llm_hbm,
             ids_out,
             idxb, maskb, lidxb, rows0, rows1, lrows0, lrows1,
             out0, out1, sem0, sem1, lsem0, lsem1, ssem):
    cid = lax.axis_index("c")
    sid = lax.axis_index("s")
    wid = sid * 2 + cid
    base = wid * BPW

    cp1 = pltpu.async_copy(tidx_hbm.at[:, pl.ds(base, BPW)], idxb, ssem)
    cp2 = pltpu.async_copy(mask_hbm.at[:, pl.ds(base, BPW)], maskb, ssem)
    cp3 = pltpu.async_copy(lidx_hbm.at[pl.ds(base, BPW)], lidxb, ssem)
    cp1.wait()
    cp2.wait()
    cp3.wait()

    rows = [rows0, rows1]
    lrows = [lrows0, lrows1]
    outs = [out0, out1]
    sems = [sem0, sem1]
    lsems = [lsem0, lsem1]

    def fire(t, b):
        for l in range(L):
            pltpu.async_copy(
                tool_hbm.at[idxb.at[l, pl.ds(t * C, C)]],
                rows[b].at[pl.ds(l * C, C)], sems[b])
        pltpu.async_copy(llm_hbm.at[lidxb.at[pl.ds(t * C, C)]],
                         lrows[b], lsems[b])

    def drain(b):
        pltpu.make_async_copy(tool_hbm.at[pl.ds(0, G)], rows[b],
                              sems[b]).wait()
        pltpu.make_async_copy(llm_hbm.at[pl.ds(0, C)], lrows[b],
                              lsems[b]).wait()

    def compute(t, b):
        def elem_body(c, carry):
            col16 = t * C + (c & ~15)
            lane = jnp.full((16,), c & 15, jnp.int32)
            msum = jnp.zeros((16,), jnp.float32)
            acc = [jnp.zeros((16,), jnp.float32) for _ in range(4)]
            for l in range(L):
                mv = maskb[l, pl.ds(col16, 16)]
                m = mv.at[lane].get(mode="promise_in_bounds")
                msum = msum + m
                r = l * C + c
                for k in range(4):
                    acc[k] = acc[k] + m * rows[b][r, pl.ds(k * 16, 16)]
            inv = 1.0 / (msum + 1e-8)
            for k in range(4):
                outs[b][c, pl.ds(k * 16, 16)] = acc[k] * inv
            return carry

        lax.fori_loop(0, C, elem_body, 0)
        eb = base + t * C
        pltpu.sync_copy(lrows[b],
                        ids_out.at[pl.ds(eb, C), pl.ds(0, ID_DIM)])
        pltpu.sync_copy(outs[b],
                        ids_out.at[pl.ds(eb, C), pl.ds(ID_DIM, ID_DIM)])

    fire(0, 0)

    def pair_body(g2, carry):
        t0 = 2 * g2
        fire(t0 + 1, 1)
        drain(0)
        compute(t0, 0)

        @pl.when(g2 < NCH // 2 - 1)
        def _():
            fire(t0 + 2, 0)

        drain(1)
        compute(t0 + 1, 1)
        return carry

    lax.fori_loop(0, NCH // 2, pair_body, 0)


@jax.jit
def _sc_call(tidx_flat, mask_flat, lidx, emb_tool, emb_llm):
    mesh = plsc.VectorSubcoreMesh(core_axis_name="c", subcore_axis_name="s")
    fn = functools.partial(
        pl.kernel,
        mesh=mesh,
        compiler_params=pltpu.CompilerParams(use_tc_tiling_on_sc=False),
        out_type=jax.ShapeDtypeStruct((B, 2 * ID_DIM), jnp.float32),
        scratch_types=[
            pltpu.VMEM((L, BPW), jnp.int32),
            pltpu.VMEM((L, BPW), jnp.float32),
            pltpu.VMEM((BPW,), jnp.int32),
            pltpu.VMEM((G, ID_DIM), jnp.float32),
            pltpu.VMEM((G, ID_DIM), jnp.float32),
            pltpu.VMEM((C, ID_DIM), jnp.float32),
            pltpu.VMEM((C, ID_DIM), jnp.float32),
            pltpu.VMEM((C, ID_DIM), jnp.float32),
            pltpu.VMEM((C, ID_DIM), jnp.float32),
            pltpu.SemaphoreType.DMA,
            pltpu.SemaphoreType.DMA,
            pltpu.SemaphoreType.DMA,
            pltpu.SemaphoreType.DMA,
            pltpu.SemaphoreType.DMA,
        ],
    )(_sc_body)
    return fn(tidx_flat, mask_flat, lidx, emb_tool, emb_llm)


def _mm_body(a_ref, wc_ref, o_ref):
    o_ref[...] = jnp.dot(a_ref[...], wc_ref[...],
                         preferred_element_type=jnp.float32)


@jax.jit
def _mm_call(a, wc_t):
    R = 2048
    return pl.pallas_call(
        _mm_body,
        grid=(B // R,),
        in_specs=[
            pl.BlockSpec((R, D_CONTENT), lambda i: (i, 0)),
            pl.BlockSpec((D_CONTENT, TOK_DIM), lambda i: (0, 0)),
        ],
        out_specs=pl.BlockSpec((R, TOK_DIM), lambda i: (i, 0)),
        out_shape=jax.ShapeDtypeStruct((B, TOK_DIM), jnp.float32),
    )(a, wc_t)


def _fin_body(e1_ref, ids_ref, wi_ref, o_ref):
    e = e1_ref[...] + jnp.dot(ids_ref[...], wi_ref[...],
                              preferred_element_type=jnp.float32)
    n = jnp.sqrt(jnp.sum(e * e, axis=1, keepdims=True))
    o_ref[...] = e / jnp.maximum(n, 1e-12)


@jax.jit
def _fin_call(e1, ids, wi_t):
    R = 2048
    return pl.pallas_call(
        _fin_body,
        grid=(B // R,),
        in_specs=[
            pl.BlockSpec((R, TOK_DIM), lambda i: (i, 0)),
            pl.BlockSpec((R, 2 * ID_DIM), lambda i: (i, 0)),
            pl.BlockSpec((2 * ID_DIM, TOK_DIM), lambda i: (0, 0)),
        ],
        out_specs=pl.BlockSpec((R, TOK_DIM), lambda i: (i, 0)),
        out_shape=jax.ShapeDtypeStruct((B, TOK_DIM), jnp.float32),
    )(e1, ids, wi_t)


def kernel(A_content, tool_idx_pad, tool_mask, llm_idx, emb_tool, emb_llm,
           W_content, W_ids):
    tidx_t = tool_idx_pad.astype(jnp.int32).T
    mask_t = tool_mask.T
    lidx = llm_idx.astype(jnp.int32)
    ids = _sc_call(tidx_t, mask_t, lidx, emb_tool, emb_llm)
    e1 = _mm_call(A_content, W_content.T)
    return _fin_call(e1, ids, W_ids.T)
